# Initial kernel scaffold; baseline (speedup 1.0000x reference)
import jax
import jax.numpy as jnp
from jax.experimental import pallas as pl


def kernel(token_ids, weight):
    raise NotImplementedError("write your pallas kernel here")


# speedup vs baseline: 1.5474x; 1.5474x over previous
import functools

import jax
import jax.numpy as jnp
from jax import lax
from jax.experimental import pallas as pl
from jax.experimental.pallas import tpu as pltpu
from jax.experimental.pallas import tpu_sc as plsc


@functools.cache
def _make_gather(V, D, N):
    info = plsc.get_sparse_core_info()
    NC, NS = info.num_cores, info.num_subcores
    NW = NC * NS
    assert N % NW == 0
    b_per_w = N // NW
    CH = 1024
    assert b_per_w % CH == 0
    n_ch = b_per_w // CH
    mesh = plsc.VectorSubcoreMesh(core_axis_name="c", subcore_axis_name="s")

    @functools.partial(
        pl.kernel,
        mesh=mesh,
        out_type=jax.ShapeDtypeStruct((N, D), jnp.float32),
        scratch_types=[
            pltpu.VMEM((CH,), jnp.int32),
            pltpu.VMEM((CH, D), jnp.float32),
            pltpu.SemaphoreType.DMA,
        ],
        compiler_params=pltpu.CompilerParams(use_tc_tiling_on_sc=False),
    )
    def gather_kernel(idx_hbm, table_hbm, out_hbm, idx_v, rows_v, sem):
        wid = lax.axis_index("s") * NC + lax.axis_index("c")
        base = wid * b_per_w

        def body(i, carry):
            off = base + i * CH
            pltpu.sync_copy(idx_hbm.at[pl.ds(off, CH)], idx_v)
            pltpu.async_copy(table_hbm.at[idx_v], rows_v, sem).wait()
            pltpu.sync_copy(rows_v, out_hbm.at[pl.ds(off, CH)])
            return carry

        lax.fori_loop(0, n_ch, body, 0)

    return gather_kernel


def kernel(token_ids, weight):
    B, F = token_ids.shape
    V, D = weight.shape
    N = B * F
    idx = token_ids.reshape(N)
    out = _make_gather(V, D, N)(idx, weight)
    return out.reshape(B, F, D)


# speedup vs baseline: 1.5773x; 1.0193x over previous
import functools

import jax
import jax.numpy as jnp
from jax import lax
from jax.experimental import pallas as pl
from jax.experimental.pallas import tpu as pltpu
from jax.experimental.pallas import tpu_sc as plsc

_CH = 1024
_NBUF = 3


@functools.cache
def _make_gather(V, D, N):
    info = plsc.get_sparse_core_info()
    NC, NS = info.num_cores, info.num_subcores
    NW = NC * NS
    assert N % (NW * _CH) == 0
    b_per_w = N // NW
    n_ch = b_per_w // _CH
    mesh = plsc.VectorSubcoreMesh(core_axis_name="c", subcore_axis_name="s")

    @functools.partial(
        pl.kernel,
        mesh=mesh,
        out_type=jax.ShapeDtypeStruct((N, D), jnp.float32),
        scratch_types=[
            pltpu.VMEM((n_ch, _CH), jnp.int32),
            pltpu.VMEM((_NBUF, _CH, D), jnp.float32),
        ]
        + [pltpu.SemaphoreType.DMA] * (2 * _NBUF),
        compiler_params=pltpu.CompilerParams(use_tc_tiling_on_sc=False),
    )
    def gather_kernel(idx_hbm, table_hbm, out_hbm, idx_all, rows, *sems):
        sem_g, sem_w = sems[:_NBUF], sems[_NBUF:]
        wid = lax.axis_index("s") * NC + lax.axis_index("c")
        base = wid * b_per_w
        pltpu.sync_copy(idx_hbm.at[wid], idx_all)

        gathers = {}
        for b in range(min(_NBUF, n_ch)):
            gathers[b] = pltpu.async_copy(
                table_hbm.at[idx_all.at[b]], rows.at[b], sem_g[b])
        for i in range(n_ch):
            b = i % _NBUF
            gathers[i].wait()
            wb = pltpu.async_copy(
                rows.at[b], out_hbm.at[pl.ds(base + i * _CH, _CH)], sem_w[b])
            nxt = i + _NBUF
            wb.wait()
            if nxt < n_ch:
                gathers[nxt] = pltpu.async_copy(
                    table_hbm.at[idx_all.at[nxt]], rows.at[b], sem_g[b])

    return gather_kernel


def kernel(token_ids, weight):
    B, F = token_ids.shape
    V, D = weight.shape
    N = B * F
    info = plsc.get_sparse_core_info()
    NW = info.num_cores * info.num_subcores
    idx = token_ids.reshape(NW, N // (NW * _CH), _CH)
    out = _make_gather(V, D, N)(idx, weight)
    return out.reshape(B, F, D)
